# Initial kernel scaffold; baseline (speedup 1.0000x reference)
import jax
import jax.numpy as jnp
from jax.experimental import pallas as pl


def kernel(x, codebook):
    raise NotImplementedError("write your pallas kernel here")


# speedup vs baseline: 1.5379x; 1.5379x over previous
import functools

import jax
import jax.numpy as jnp
from jax.experimental import pallas as pl
from jax.experimental.pallas import tpu as pltpu

_N = 32768
_K = 1024
_D = 64
_BN = 1024
_GRID = _N // _BN


def _vq_body(x_ref, xsq_ref, cbt2_ref, cbsq_ref, cb_ref,
             codes_ref, quant_ref, loss_ref):
    i = pl.program_id(0)

    x = x_ref[...].reshape(_BN, _D)
    dot2 = jnp.dot(x, cbt2_ref[...], preferred_element_type=jnp.float32)
    dist = (xsq_ref[...] - dot2) + cbsq_ref[...]

    minval = jnp.min(dist, axis=-1, keepdims=True)
    col = jax.lax.broadcasted_iota(jnp.int32, (_BN, _K), 1)
    cand = jnp.where(dist == minval, col, jnp.int32(_K))
    code = jnp.min(cand, axis=-1)
    codes_ref[...] = code.reshape(1, 1, _BN)

    onehot = (cand == code[:, None]).astype(jnp.float32)
    q = jnp.dot(onehot, cb_ref[...], preferred_element_type=jnp.float32)

    d = q - x
    quant_ref[...] = (x + d).astype(jnp.bfloat16).reshape(1, _BN, _D)

    part = jnp.sum(d * d)

    @pl.when(i == 0)
    def _():
        loss_ref[0, 0] = 0.0

    loss_ref[0, 0] += part


@jax.jit
def kernel(x, codebook):
    x_flat = x.reshape(-1, _D).astype(jnp.float32)
    cb = codebook.astype(jnp.float32)
    x_sq = jnp.sum(x_flat ** 2, axis=-1, keepdims=True)
    cb_sq = jnp.sum(cb ** 2, axis=-1).reshape(1, _K)
    cbt2 = (cb + cb).T

    grid = (_GRID,)
    codes, quant, loss_sum = pl.pallas_call(
        _vq_body,
        grid=grid,
        in_specs=[
            pl.BlockSpec((1, _BN, _D), lambda i: (i, 0, 0)),
            pl.BlockSpec((_BN, 1), lambda i: (i, 0)),
            pl.BlockSpec((_D, _K), lambda i: (0, 0)),
            pl.BlockSpec((1, _K), lambda i: (0, 0)),
            pl.BlockSpec((_K, _D), lambda i: (0, 0)),
        ],
        out_specs=[
            pl.BlockSpec((1, 1, _BN), lambda i: (i, 0, 0)),
            pl.BlockSpec((1, _BN, _D), lambda i: (i, 0, 0)),
            pl.BlockSpec(memory_space=pltpu.SMEM, block_shape=(1, 1),
                         index_map=lambda i: (0, 0)),
        ],
        out_shape=[
            jax.ShapeDtypeStruct((_GRID, 1, _BN), jnp.int32),
            jax.ShapeDtypeStruct((_GRID, _BN, _D), jnp.bfloat16),
            jax.ShapeDtypeStruct((1, 1), jnp.float32),
        ],
    )(x.reshape(_GRID, _BN, _D).astype(jnp.float32), x_sq, cbt2, cb_sq, cb)

    loss = loss_sum[0, 0] / jnp.float32(_N * _D)
    quantized = quant.reshape(x.shape)
    codes_out = codes.reshape(x.shape[:-1])
    return (quantized, codes_out, loss, loss)
